# baseline (speedup 1.0000x reference)
import jax
import jax.numpy as jnp
from jax.experimental import pallas as pl
from jax.experimental.pallas import tpu as pltpu

WINDOW = 50
BATCH = 4096
DIM = 128


def _body(ptr_ref, x_ref, data_ref, out_ref, sem):
    p = ptr_ref[0]
    for i in range(WINDOW):
        @pl.when(i != p)
        def _copy():
            pltpu.make_async_copy(data_ref.at[i], out_ref.at[i], sem).start()

        @pl.when(i == p)
        def _overwrite():
            pltpu.make_async_copy(x_ref, out_ref.at[i], sem).start()
    for i in range(WINDOW):
        pltpu.make_async_copy(data_ref.at[i], out_ref.at[i], sem).wait()


def kernel(x, data, pointer):
    ptr = jnp.atleast_1d(jnp.asarray(pointer, dtype=jnp.int32))
    grid_spec = pltpu.PrefetchScalarGridSpec(
        num_scalar_prefetch=1,
        grid=(1,),
        in_specs=[
            pl.BlockSpec(memory_space=pl.MemorySpace.ANY),
            pl.BlockSpec(memory_space=pl.MemorySpace.ANY),
        ],
        out_specs=pl.BlockSpec(memory_space=pl.MemorySpace.ANY),
        scratch_shapes=[pltpu.SemaphoreType.DMA],
    )
    return pl.pallas_call(
        _body,
        grid_spec=grid_spec,
        out_shape=jax.ShapeDtypeStruct((WINDOW, BATCH, DIM), jnp.float32),
    )(ptr, x, data)


# speedup vs baseline: 43.0857x; 43.0857x over previous
import jax
import jax.numpy as jnp
from jax.experimental import pallas as pl
from jax.experimental.pallas import tpu as pltpu

WINDOW = 50
BATCH = 4096
DIM = 128


def _body(ptr_ref, x_ref, data_ref, out_ref):
    i = pl.program_id(0)
    p = ptr_ref[0]

    @pl.when(i != p)
    def _copy():
        out_ref[0] = data_ref[0]

    @pl.when(i == p)
    def _overwrite():
        out_ref[0] = x_ref[...]


def kernel(x, data, pointer):
    ptr = jnp.atleast_1d(jnp.asarray(pointer, dtype=jnp.int32))
    grid_spec = pltpu.PrefetchScalarGridSpec(
        num_scalar_prefetch=1,
        grid=(WINDOW,),
        in_specs=[
            pl.BlockSpec((BATCH, DIM), lambda i, p: (0, 0)),
            pl.BlockSpec((1, BATCH, DIM), lambda i, p: (i, 0, 0)),
        ],
        out_specs=pl.BlockSpec((1, BATCH, DIM), lambda i, p: (i, 0, 0)),
    )
    return pl.pallas_call(
        _body,
        grid_spec=grid_spec,
        out_shape=jax.ShapeDtypeStruct((WINDOW, BATCH, DIM), jnp.float32),
        compiler_params=pltpu.CompilerParams(
            dimension_semantics=("parallel",),
        ),
    )(ptr, x, data)


# speedup vs baseline: 48.0592x; 1.1154x over previous
import jax
import jax.numpy as jnp
from jax.experimental import pallas as pl
from jax.experimental.pallas import tpu as pltpu

WINDOW = 50
BATCH = 4096
DIM = 128
SLOTS = 5


def _body(ptr_ref, x_ref, data_ref, out_ref):
    i = pl.program_id(0)
    p = ptr_ref[0]
    out_ref[...] = data_ref[...]

    @pl.when(i == p // SLOTS)
    def _overwrite():
        out_ref[pl.ds((p % SLOTS) * BATCH, BATCH), :] = x_ref[...]


def kernel(x, data, pointer):
    ptr = jnp.atleast_1d(jnp.asarray(pointer, dtype=jnp.int32))
    flat = data.reshape(WINDOW * BATCH, DIM)
    grid_spec = pltpu.PrefetchScalarGridSpec(
        num_scalar_prefetch=1,
        grid=(WINDOW // SLOTS,),
        in_specs=[
            pl.BlockSpec((BATCH, DIM), lambda i, p: (0, 0)),
            pl.BlockSpec((SLOTS * BATCH, DIM), lambda i, p: (i, 0)),
        ],
        out_specs=pl.BlockSpec((SLOTS * BATCH, DIM), lambda i, p: (i, 0)),
    )
    out = pl.pallas_call(
        _body,
        grid_spec=grid_spec,
        out_shape=jax.ShapeDtypeStruct((WINDOW * BATCH, DIM), jnp.float32),
        compiler_params=pltpu.CompilerParams(
            dimension_semantics=("arbitrary",),
        ),
    )(ptr, x, flat)
    return out.reshape(WINDOW, BATCH, DIM)
